# baseline (speedup 1.0000x reference)
import functools

import jax
import jax.numpy as jnp
from jax import lax
from jax.experimental import pallas as pl
from jax.experimental.pallas import tpu as pltpu
from jax.experimental.pallas import tpu_sc as plsc

N = 16384
NC, NS, L = 2, 16, 16
NW = NC * NS
CHUNK = N // NW
BLK = 128
NBLK = CHUNK // BLK
HALF = NBLK // 2


@functools.partial(
    pl.kernel,
    mesh=plsc.VectorSubcoreMesh(core_axis_name="c", subcore_axis_name="s"),
    out_type=jax.ShapeDtypeStruct((N,), jnp.float32),
    compiler_params=pltpu.CompilerParams(needs_layout_passes=False),
    scratch_types=[
        pltpu.VMEM((NBLK, 2, BLK), jnp.float32),
        pltpu.VMEM((NBLK, 2, BLK), jnp.float32),
        pltpu.VMEM((8, 8), jnp.float32),
        pltpu.VMEM((8,), jnp.float32),
        pltpu.VMEM((CHUNK,), jnp.float32),
        pltpu.SemaphoreType.DMA,
        pltpu.SemaphoreType.DMA,
        pltpu.SemaphoreType.DMA,
        pltpu.SemaphoreType.DMA,
    ],
)
def _sc_kernel(x_hbm, y_hbm, alphas_hbm, beta_hbm, out_hbm,
               x_v, y_v, a_v, b_v, o_v, sem_t, sem_h0, sem_h1, sem_o):
    wid = lax.axis_index("s") * NC + lax.axis_index("c")
    base = wid * NBLK
    tables = [
        pltpu.make_async_copy(alphas_hbm, a_v, sem_t),
        pltpu.make_async_copy(beta_hbm, b_v, sem_t),
    ]
    halves = []
    for h, sem in ((0, sem_h0), (1, sem_h1)):
        hs = pl.ds(h * HALF, HALF)
        halves.append([
            pltpu.make_async_copy(x_hbm.at[pl.ds(base + h * HALF, HALF)],
                                  x_v.at[hs], sem),
            pltpu.make_async_copy(y_hbm.at[pl.ds(base + h * HALF, HALF)],
                                  y_v.at[hs], sem),
        ])
    for c in tables + halves[0] + halves[1]:
        c.start()
    for c in tables:
        c.wait()

    def step(j):
        blk = j >> 3
        k = (j & 7) * L
        s = pl.ds(k, L)
        t_x = x_v[blk, 0, s]
        t_y = y_v[blk, 0, s]
        xi = x_v[blk, 1, s].astype(jnp.int32)
        yi = y_v[blk, 1, s].astype(jnp.int32)
        a = plsc.load_gather(a_v, [xi, yi])
        b = plsc.load_gather(b_v, [yi])
        o_v[pl.ds(blk * BLK + k, L)] = jnp.where(
            t_x > 0.0, a * b * jnp.exp(b * (t_y - t_x)), 0.0)

    outs = []
    steps_per_half = HALF * BLK // L
    for h, sem in ((0, sem_h0), (1, sem_h1)):
        for c in halves[h]:
            c.wait()
        plsc.parallel_loop(h * steps_per_half, (h + 1) * steps_per_half, 1,
                           unroll=2)(step)
        span = pl.ds(h * HALF * BLK, HALF * BLK)
        out_c = pltpu.make_async_copy(
            o_v.at[span], out_hbm.at[pl.ds(wid * CHUNK + h * HALF * BLK,
                                           HALF * BLK)], sem_o)
        out_c.start()
        outs.append(out_c)
    for c in outs:
        c.wait()


def kernel(x, y, alphas, beta):
    xb = jnp.swapaxes(x.reshape(BLK, BLK, 2), 1, 2)
    yb = jnp.swapaxes(y.reshape(BLK, BLK, 2), 1, 2)
    return _sc_kernel(xb, yb, alphas, beta)
